# Initial kernel scaffold; baseline (speedup 1.0000x reference)
import jax
import jax.numpy as jnp
from jax.experimental import pallas as pl


def kernel(x, table):
    raise NotImplementedError("write your pallas kernel here")


# speedup vs baseline: 2.0904x; 2.0904x over previous
import functools

import jax
import jax.numpy as jnp
from jax import lax
from jax.experimental import pallas as pl
from jax.experimental.pallas import tpu as pltpu
from jax.experimental.pallas import tpu_sc as plsc

B = 1024
L = 200
D = 64
NC = 2
NS = 16
NW = NC * NS
SENT_PER_W = B // NW
HALF = L // 2
LANES = 16


def _positional_encoding(max_seq_len, d_model):
    even_i = jnp.arange(0, d_model, 2, dtype=jnp.float32)
    denominator = jnp.power(10000.0, even_i / d_model)
    pos = jnp.arange(max_seq_len, dtype=jnp.float32).reshape(max_seq_len, 1)
    even_pe = jnp.sin(pos / denominator)
    odd_pe = jnp.cos(pos / denominator)
    stacked = jnp.stack([even_pe, odd_pe], axis=2)
    return stacked.reshape(max_seq_len, d_model)


def _make_sc_call():
    mesh = plsc.VectorSubcoreMesh(core_axis_name="c", subcore_axis_name="s")

    @functools.partial(
        pl.kernel,
        mesh=mesh,
        out_type=jax.ShapeDtypeStruct((B * L, D), jnp.float32),
        compiler_params=pltpu.CompilerParams(use_tc_tiling_on_sc=False),
        scratch_types=[
            pltpu.VMEM((2 * SENT_PER_W, HALF), jnp.int32),
            pltpu.VMEM((L, D), jnp.float32),
            pltpu.VMEM((L, D), jnp.float32),
            pltpu.SemaphoreType.DMA,
        ],
    )
    def sc_embed(table_h, idx_h, pe_h, out_h, idx_v, pe_v, gbuf, sem):
        wid = lax.axis_index("s") * NC + lax.axis_index("c")
        pltpu.sync_copy(idx_h.at[wid], idx_v)
        pltpu.sync_copy(pe_h, pe_v)

        def per_sentence(s, carry):
            c0 = pltpu.async_copy(
                table_h.at[idx_v.at[2 * s]], gbuf.at[pl.ds(0, HALF)], sem)
            c1 = pltpu.async_copy(
                table_h.at[idx_v.at[2 * s + 1]], gbuf.at[pl.ds(HALF, HALF)], sem)
            c0.wait()
            c1.wait()

            def per_row(r, c):
                for j in range(D // LANES):
                    sl = pl.ds(j * LANES, LANES)
                    gbuf[r, sl] = gbuf[r, sl] + pe_v[r, sl]
                return c

            lax.fori_loop(0, L, per_row, 0, unroll=2)
            base = wid * (SENT_PER_W * L) + s * L
            pltpu.sync_copy(gbuf, out_h.at[pl.ds(base, L)])
            return carry

        lax.fori_loop(0, SENT_PER_W, per_sentence, 0)

    return sc_embed


_sc_embed = _make_sc_call()


def kernel(x, table):
    pe = _positional_encoding(L, D)
    idx3 = x.reshape(NW, 2 * SENT_PER_W, HALF)
    out = _sc_embed(table, idx3, pe)
    return out.reshape(B, L, D)


# speedup vs baseline: 3.2050x; 1.5332x over previous
import functools

import jax
import jax.numpy as jnp
from jax import lax
from jax.experimental import pallas as pl
from jax.experimental.pallas import tpu as pltpu
from jax.experimental.pallas import tpu_sc as plsc

B = 1024
L = 200
D = 64
NC = 2
NS = 16
NW = NC * NS
SENT_PER_W = B // NW
HALF = L // 2
LANES = 16


def _positional_encoding(max_seq_len, d_model):
    even_i = jnp.arange(0, d_model, 2, dtype=jnp.float32)
    denominator = jnp.power(10000.0, even_i / d_model)
    pos = jnp.arange(max_seq_len, dtype=jnp.float32).reshape(max_seq_len, 1)
    even_pe = jnp.sin(pos / denominator)
    odd_pe = jnp.cos(pos / denominator)
    stacked = jnp.stack([even_pe, odd_pe], axis=2)
    return stacked.reshape(max_seq_len, d_model)


def _make_sc_call():
    mesh = plsc.VectorSubcoreMesh(core_axis_name="c", subcore_axis_name="s")

    @functools.partial(
        pl.kernel,
        mesh=mesh,
        out_type=jax.ShapeDtypeStruct((B * L, D), jnp.float32),
        compiler_params=pltpu.CompilerParams(use_tc_tiling_on_sc=False),
        scratch_types=[
            pltpu.VMEM((2 * SENT_PER_W, HALF), jnp.int32),
            pltpu.VMEM((L, D), jnp.float32),
            pltpu.VMEM((L, D), jnp.float32),
            pltpu.VMEM((L, D), jnp.float32),
            pltpu.VMEM((L, D), jnp.float32),
            pltpu.VMEM((L, D), jnp.float32),
            pltpu.SemaphoreType.DMA,
            pltpu.SemaphoreType.DMA,
            pltpu.SemaphoreType.DMA,
            pltpu.SemaphoreType.DMA,
        ],
    )
    def sc_embed(table_h, idx_h, pe_h, out_h,
                 idx_v, pe_v, g0, g1, o0, o1, gs0, gs1, ss0, ss1):
        wid = lax.axis_index("s") * NC + lax.axis_index("c")
        pltpu.sync_copy(idx_h.at[wid], idx_v)
        pltpu.sync_copy(pe_h, pe_v)
        row0 = wid * (SENT_PER_W * L)

        gbufs, obufs = (g0, g1), (o0, o1)
        gsems, ssems = (gs0, gs1), (ss0, ss1)

        def fire_gather(s, gbuf, gsem):
            pltpu.async_copy(
                table_h.at[idx_v.at[2 * s]], gbuf.at[pl.ds(0, HALF)], gsem)
            pltpu.async_copy(
                table_h.at[idx_v.at[2 * s + 1]], gbuf.at[pl.ds(HALF, HALF)],
                gsem)

        def wait_gather(s, gbuf, gsem):
            pltpu.make_async_copy(
                table_h.at[idx_v.at[2 * s]], gbuf.at[pl.ds(0, HALF)],
                gsem).wait()
            pltpu.make_async_copy(
                table_h.at[idx_v.at[2 * s + 1]], gbuf.at[pl.ds(HALF, HALF)],
                gsem).wait()

        def wait_scatter(obuf, ssem):
            pltpu.make_async_copy(obuf, out_h.at[pl.ds(row0, L)], ssem).wait()

        fire_gather(0, g0, gs0)
        fire_gather(1, g1, gs1)

        @pl.loop(0, SENT_PER_W, step=2)
        def per_pair(s0):
            for b in range(2):
                s = s0 + b
                gbuf, obuf = gbufs[b], obufs[b]
                gsem, ssem = gsems[b], ssems[b]
                wait_gather(s, gbuf, gsem)

                @pl.when(s >= 2)
                def _():
                    wait_scatter(obuf, ssem)

                @plsc.parallel_loop(0, L, unroll=4)
                def per_row(r):
                    for j in range(D // LANES):
                        sl = pl.ds(j * LANES, LANES)
                        obuf[r, sl] = gbuf[r, sl] + pe_v[r, sl]

                @pl.when(s + 2 < SENT_PER_W)
                def _():
                    fire_gather(s + 2, gbuf, gsem)

                pltpu.async_copy(
                    obuf, out_h.at[pl.ds(row0 + s * L, L)], ssem)

        wait_scatter(o0, ss0)
        wait_scatter(o1, ss1)

    return sc_embed


_sc_embed = _make_sc_call()


def kernel(x, table):
    pe = _positional_encoding(L, D)
    idx3 = x.reshape(NW, 2 * SENT_PER_W, HALF)
    out = _sc_embed(table, idx3, pe)
    return out.reshape(B, L, D)
